# baseline (speedup 1.0000x reference)
import functools

import jax
import jax.numpy as jnp
from jax import lax
from jax.experimental import pallas as pl
from jax.experimental.pallas import tpu as pltpu
from jax.experimental.pallas import tpu_sc as plsc

EMB = 128
VOCAB = 27
PEP = 31
PEPP = 32
BATCH = 16384
NC, NS = 2, 16
NW = NC * NS
BPW = BATCH // NW
KB = 4
KROW = KB * PEPP
NCHUNK = BPW // KB
NBUF = 4
OROWS = BATCH * PEPP


def _prep_body(x_ref, aa_ref, pos_ref, comb_ref, idx_ref):
    comb_ref[...] = aa_ref[...][:, None, :] + pos_ref[...][None, :, :]
    p = lax.broadcasted_iota(jnp.int32, (BATCH, PEP), 1)
    idx_ref[...] = jnp.concatenate(
        [x_ref[...] * PEP + p, jnp.zeros((BATCH, 1), jnp.int32)], axis=1)


def _prep(x, aa_table, pos_table):
    return pl.pallas_call(
        _prep_body,
        out_shape=(
            jax.ShapeDtypeStruct((VOCAB, PEP, EMB), jnp.float32),
            jax.ShapeDtypeStruct((BATCH, PEPP), jnp.int32),
        ),
    )(x, aa_table, pos_table)


def _sc_gather(comb, idx3):
    mesh = plsc.VectorSubcoreMesh(core_axis_name="c", subcore_axis_name="s")

    @functools.partial(
        pl.kernel,
        mesh=mesh,
        out_type=jax.ShapeDtypeStruct((OROWS, EMB), jnp.float32),
        scratch_types=[
            pltpu.VMEM((NCHUNK, KROW), jnp.int32),
            *[pltpu.VMEM((KROW, EMB), jnp.float32) for _ in range(NBUF)],
            pltpu.SemaphoreType.DMA((NBUF,)),
            pltpu.SemaphoreType.DMA((NBUF,)),
        ],
    )
    def k(comb_hbm, idx_hbm, out_hbm, idx_all, r0, r1, r2, r3, gsem, osem):
        rows = [r0, r1, r2, r3]
        wid = lax.axis_index("s") * NC + lax.axis_index("c")
        base = wid * NCHUNK * KROW
        pltpu.sync_copy(idx_hbm.at[wid], idx_all)

        def wait_gather(s):
            pltpu.make_async_copy(
                comb_hbm.at[idx_all.at[0]], rows[s], gsem.at[s]).wait()

        def wait_out(s):
            pltpu.make_async_copy(
                rows[s], out_hbm.at[pl.ds(base, KROW)], osem.at[s]).wait()

        def phys(j):
            jj = j + wid * NBUF
            return jnp.where(jj >= NCHUNK, jj - NCHUNK, jj)

        def start_gather(j, s):
            pltpu.async_copy(
                comb_hbm.at[idx_all.at[phys(j)]], rows[s], gsem.at[s])

        def start_out(j, s):
            pltpu.async_copy(
                rows[s], out_hbm.at[pl.ds(base + phys(j) * KROW, KROW)],
                osem.at[s])

        def body(g, _):
            for s in range(NBUF):
                j = g * NBUF + s
                pl.when(g > 0)(lambda s=s: wait_out(s))
                start_gather(j, s)
                ps = (s - 1) % NBUF
                if s == 0:
                    def prev(g=g, ps=ps):
                        wait_gather(ps)
                        start_out(g * NBUF - 1, ps)
                    pl.when(g > 0)(prev)
                else:
                    wait_gather(ps)
                    start_out(j - 1, ps)
            return 0

        lax.fori_loop(0, NCHUNK // NBUF, body, 0)
        wait_gather(NBUF - 1)
        start_out(NCHUNK - 1, NBUF - 1)
        for s in range(NBUF):
            wait_out(s)

    return k(comb, idx3)


def kernel(x, aa_table, pos_table):
    x32 = x.astype(jnp.int32)
    comb3, idx = _prep(x32, aa_table, pos_table)
    comb = comb3.reshape(VOCAB * PEP, EMB)
    out = _sc_gather(comb, idx.reshape(NW, NCHUNK, KROW))
    return out.reshape(BATCH, PEPP, EMB)[:, :PEP, :]


# speedup vs baseline: 1.7699x; 1.7699x over previous
import functools

import jax
import jax.numpy as jnp
from jax import lax
from jax.experimental import pallas as pl
from jax.experimental.pallas import tpu as pltpu
from jax.experimental.pallas import tpu_sc as plsc

EMB = 128
VOCAB = 27
PEP = 31
BATCH = 16384
ROWS = BATCH * PEP
NC, NS = 2, 16
NW = NC * NS
KROW = 128
NCHUNK = ROWS // NW // KROW
NBUF = 4
OROWS = ROWS


def _prep_body(x_ref, aa_ref, pos_ref, comb_ref, idx_ref):
    comb_ref[...] = aa_ref[...][:, None, :] + pos_ref[...][None, :, :]
    p = lax.broadcasted_iota(jnp.int32, (BATCH, PEP), 1)
    idx_ref[...] = x_ref[...] * PEP + p


def _prep(x, aa_table, pos_table):
    return pl.pallas_call(
        _prep_body,
        out_shape=(
            jax.ShapeDtypeStruct((VOCAB, PEP, EMB), jnp.float32),
            jax.ShapeDtypeStruct((BATCH, PEP), jnp.int32),
        ),
    )(x, aa_table, pos_table)


def _sc_gather(comb, idx3):
    mesh = plsc.VectorSubcoreMesh(core_axis_name="c", subcore_axis_name="s")

    @functools.partial(
        pl.kernel,
        mesh=mesh,
        out_type=jax.ShapeDtypeStruct((OROWS, EMB), jnp.float32),
        scratch_types=[
            pltpu.VMEM((NCHUNK, KROW), jnp.int32),
            *[pltpu.VMEM((KROW, EMB), jnp.float32) for _ in range(NBUF)],
            pltpu.SemaphoreType.DMA((NBUF,)),
            pltpu.SemaphoreType.DMA((NBUF,)),
        ],
    )
    def k(comb_hbm, idx_hbm, out_hbm, idx_all, r0, r1, r2, r3, gsem, osem):
        rows = [r0, r1, r2, r3]
        wid = lax.axis_index("s") * NC + lax.axis_index("c")
        base = wid * NCHUNK * KROW
        pltpu.sync_copy(idx_hbm.at[wid], idx_all)

        def wait_gather(s):
            pltpu.make_async_copy(
                comb_hbm.at[idx_all.at[0]], rows[s], gsem.at[s]).wait()

        def wait_out(s):
            pltpu.make_async_copy(
                rows[s], out_hbm.at[pl.ds(base, KROW)], osem.at[s]).wait()

        def phys(j):
            jj = j + wid * NBUF
            return jnp.where(jj >= NCHUNK, jj - NCHUNK, jj)

        def start_gather(j, s):
            pltpu.async_copy(
                comb_hbm.at[idx_all.at[phys(j)]], rows[s], gsem.at[s])

        def start_out(j, s):
            pltpu.async_copy(
                rows[s], out_hbm.at[pl.ds(base + phys(j) * KROW, KROW)],
                osem.at[s])

        def body(g, _):
            for s in range(NBUF):
                j = g * NBUF + s
                pl.when(g > 0)(lambda s=s: wait_out(s))
                start_gather(j, s)
                ps = (s - 1) % NBUF
                if s == 0:
                    def prev(g=g, ps=ps):
                        wait_gather(ps)
                        start_out(g * NBUF - 1, ps)
                    pl.when(g > 0)(prev)
                else:
                    wait_gather(ps)
                    start_out(j - 1, ps)
            return 0

        lax.fori_loop(0, NCHUNK // NBUF, body, 0)
        wait_gather(NBUF - 1)
        start_out(NCHUNK - 1, NBUF - 1)
        for s in range(NBUF):
            wait_out(s)

    return k(comb, idx3)


def kernel(x, aa_table, pos_table):
    x32 = x.astype(jnp.int32)
    comb3, idx = _prep(x32, aa_table, pos_table)
    comb = comb3.reshape(VOCAB * PEP, EMB)
    out = _sc_gather(comb, idx.reshape(NW, NCHUNK, KROW))
    return out.reshape(BATCH, PEP, EMB)


# speedup vs baseline: 2.4055x; 1.3591x over previous
import functools

import jax
import jax.numpy as jnp
from jax import lax
from jax.experimental import pallas as pl
from jax.experimental.pallas import tpu as pltpu
from jax.experimental.pallas import tpu_sc as plsc

EMB = 128
VOCAB = 27
PEP = 31
BATCH = 16384
ROWS = BATCH * PEP
NC, NS = 2, 16
NW = NC * NS
KROW = 128
NCHUNK = ROWS // NW // KROW
NBUF = 4
OROWS = ROWS


def _prep_body(x_ref, aa_ref, pos_ref, comb_ref, idx_ref):
    comb_ref[...] = aa_ref[...][:, None, :] + pos_ref[...][None, :, :]
    p = lax.broadcasted_iota(jnp.int32, (BATCH, PEP), 1)
    idx_ref[...] = x_ref[...] * PEP + p


def _prep(x, aa_table, pos_table):
    return pl.pallas_call(
        _prep_body,
        out_shape=(
            jax.ShapeDtypeStruct((VOCAB, PEP, EMB), jnp.float32),
            jax.ShapeDtypeStruct((BATCH, PEP), jnp.int32),
        ),
    )(x, aa_table, pos_table)


def _sc_gather(comb, idx3):
    mesh = plsc.VectorSubcoreMesh(core_axis_name="c", subcore_axis_name="s")

    @functools.partial(
        pl.kernel,
        mesh=mesh,
        out_type=jax.ShapeDtypeStruct((OROWS, EMB), jnp.float32),
        scratch_types=[
            pltpu.VMEM((NCHUNK, KROW), jnp.int32),
            *[pltpu.VMEM((KROW, EMB), jnp.float32) for _ in range(NBUF)],
            pltpu.VMEM_SHARED((VOCAB * PEP, EMB), jnp.float32),
            pltpu.SemaphoreType.DMA((NBUF,)),
            pltpu.SemaphoreType.DMA((NBUF,)),
        ],
    )
    def k(comb_hbm, idx_hbm, out_hbm, idx_all, r0, r1, r2, r3, comb_sp,
          gsem, osem):
        rows = [r0, r1, r2, r3]
        wid = lax.axis_index("s") * NC + lax.axis_index("c")
        base = wid * NCHUNK * KROW
        pl.when(lax.axis_index("s") == 0)(
            lambda: pltpu.sync_copy(comb_hbm, comb_sp))
        pltpu.sync_copy(idx_hbm.at[wid], idx_all)
        plsc.subcore_barrier()

        def wait_gather(s):
            pltpu.make_async_copy(
                comb_sp.at[idx_all.at[0]], rows[s], gsem.at[s]).wait()

        def wait_out(s):
            pltpu.make_async_copy(
                rows[s], out_hbm.at[pl.ds(base, KROW)], osem.at[s]).wait()

        def phys(j):
            jj = j + wid * NBUF
            return jnp.where(jj >= NCHUNK, jj - NCHUNK, jj)

        def start_gather(j, s):
            pltpu.async_copy(
                comb_sp.at[idx_all.at[phys(j)]], rows[s], gsem.at[s])

        def start_out(j, s):
            pltpu.async_copy(
                rows[s], out_hbm.at[pl.ds(base + phys(j) * KROW, KROW)],
                osem.at[s])

        def body(g, _):
            for s in range(NBUF):
                j = g * NBUF + s
                pl.when(g > 0)(lambda s=s: wait_out(s))
                start_gather(j, s)
                ps = (s - 1) % NBUF
                if s == 0:
                    def prev(g=g, ps=ps):
                        wait_gather(ps)
                        start_out(g * NBUF - 1, ps)
                    pl.when(g > 0)(prev)
                else:
                    wait_gather(ps)
                    start_out(j - 1, ps)
            return 0

        lax.fori_loop(0, NCHUNK // NBUF, body, 0)
        wait_gather(NBUF - 1)
        start_out(NCHUNK - 1, NBUF - 1)
        for s in range(NBUF):
            wait_out(s)

    return k(comb, idx3)


def kernel(x, aa_table, pos_table):
    x32 = x.astype(jnp.int32)
    comb3, idx = _prep(x32, aa_table, pos_table)
    comb = comb3.reshape(VOCAB * PEP, EMB)
    out = _sc_gather(comb, idx.reshape(NW, NCHUNK, KROW))
    return out.reshape(BATCH, PEP, EMB)


# speedup vs baseline: 2.4123x; 1.0028x over previous
import functools

import jax
import jax.numpy as jnp
from jax import lax
from jax.experimental import pallas as pl
from jax.experimental.pallas import tpu as pltpu
from jax.experimental.pallas import tpu_sc as plsc

EMB = 128
VOCAB = 27
PEP = 31
BATCH = 16384
ROWS = BATCH * PEP
NC, NS = 2, 16
NW = NC * NS
KROW = 128
NCHUNK = ROWS // NW // KROW
NBUF = 4
OROWS = ROWS


def _prep_body(x_ref, aa_ref, pos_ref, comb_ref, idx_ref):
    comb = aa_ref[...][:, None, :] + pos_ref[...][None, :, :]
    comb_ref[...] = comb.reshape(VOCAB * PEP, EMB)
    p = lax.broadcasted_iota(jnp.int32, (BATCH, PEP), 1)
    idx_ref[...] = x_ref[...] * PEP + p


def _prep(x, aa_table, pos_table):
    return pl.pallas_call(
        _prep_body,
        out_shape=(
            jax.ShapeDtypeStruct((VOCAB * PEP, EMB), jnp.float32),
            jax.ShapeDtypeStruct((BATCH, PEP), jnp.int32),
        ),
    )(x, aa_table, pos_table)


def _sc_gather(comb, idx3):
    mesh = plsc.VectorSubcoreMesh(core_axis_name="c", subcore_axis_name="s")

    @functools.partial(
        pl.kernel,
        mesh=mesh,
        out_type=jax.ShapeDtypeStruct((OROWS, EMB), jnp.float32),
        scratch_types=[
            pltpu.VMEM((NCHUNK, KROW), jnp.int32),
            *[pltpu.VMEM((KROW, EMB), jnp.float32) for _ in range(NBUF)],
            pltpu.VMEM_SHARED((VOCAB * PEP, EMB), jnp.float32),
            pltpu.SemaphoreType.DMA((NBUF,)),
            pltpu.SemaphoreType.DMA((NBUF,)),
        ],
    )
    def k(comb_hbm, idx_hbm, out_hbm, idx_all, r0, r1, r2, r3, comb_sp,
          gsem, osem):
        rows = [r0, r1, r2, r3]
        wid = lax.axis_index("s") * NC + lax.axis_index("c")
        base = wid * NCHUNK * KROW
        pl.when(lax.axis_index("s") == 0)(
            lambda: pltpu.sync_copy(comb_hbm, comb_sp))
        pltpu.sync_copy(idx_hbm.at[wid], idx_all)
        plsc.subcore_barrier()

        def wait_gather(s):
            pltpu.make_async_copy(
                comb_sp.at[idx_all.at[0]], rows[s], gsem.at[s]).wait()

        def wait_out(s):
            pltpu.make_async_copy(
                rows[s], out_hbm.at[pl.ds(base, KROW)], osem.at[s]).wait()

        def phys(j):
            jj = j + wid * NBUF
            return jnp.where(jj >= NCHUNK, jj - NCHUNK, jj)

        def start_gather(j, s):
            pltpu.async_copy(
                comb_sp.at[idx_all.at[phys(j)]], rows[s], gsem.at[s])

        def start_out(j, s):
            pltpu.async_copy(
                rows[s], out_hbm.at[pl.ds(base + phys(j) * KROW, KROW)],
                osem.at[s])

        def body(g, _):
            for s in range(NBUF):
                j = g * NBUF + s
                pl.when(g > 0)(lambda s=s: wait_out(s))
                start_gather(j, s)
                ps = (s - 1) % NBUF
                if s == 0:
                    def prev(g=g, ps=ps):
                        wait_gather(ps)
                        start_out(g * NBUF - 1, ps)
                    pl.when(g > 0)(prev)
                else:
                    wait_gather(ps)
                    start_out(j - 1, ps)
            return 0

        lax.fori_loop(0, NCHUNK // NBUF, body, 0)
        wait_gather(NBUF - 1)
        start_out(NCHUNK - 1, NBUF - 1)
        for s in range(NBUF):
            wait_out(s)

    return k(comb, idx3)


def kernel(x, aa_table, pos_table):
    x32 = x.astype(jnp.int32)
    comb, idx = _prep(x32, aa_table, pos_table)
    out = _sc_gather(comb, idx.reshape(NW, NCHUNK, KROW))
    return out.reshape(BATCH, PEP, EMB)


# speedup vs baseline: 3.9158x; 1.6233x over previous
import functools

import jax
import jax.numpy as jnp
from jax import lax
from jax.experimental import pallas as pl
from jax.experimental.pallas import tpu as pltpu
from jax.experimental.pallas import tpu_sc as plsc

EMB = 128
VOCAB = 27
PEP = 31
BATCH = 16384
ROWS = BATCH * PEP
NC, NS = 2, 16
NW = NC * NS
KROW = 4 * PEP
NCHUNK = ROWS // NW // KROW
NBUF = 4


def _prep_body(x_ref, aa_ref, pos_ref, comb_ref, idx_ref):
    comb = aa_ref[...][:, None, :] + pos_ref[...][None, :, :]
    comb_ref[...] = comb.reshape(VOCAB * PEP, EMB)
    p = lax.broadcasted_iota(jnp.int32, (BATCH, PEP), 1)
    idx_ref[...] = x_ref[...] * PEP + p


def _prep(x, aa_table, pos_table):
    return pl.pallas_call(
        _prep_body,
        out_shape=(
            jax.ShapeDtypeStruct((VOCAB * PEP, EMB), jnp.float32),
            jax.ShapeDtypeStruct((BATCH, PEP), jnp.int32),
        ),
    )(x, aa_table, pos_table)


KB = 4
BPW = BATCH // NW


def _sc_gather(comb, idx3):
    mesh = plsc.VectorSubcoreMesh(core_axis_name="c", subcore_axis_name="s")

    @functools.partial(
        pl.kernel,
        mesh=mesh,
        out_type=jax.ShapeDtypeStruct((BATCH, PEP, EMB), jnp.float32),
        scratch_types=[
            pltpu.VMEM((NCHUNK, KROW), jnp.int32),
            *[pltpu.VMEM((KB, 32, EMB), jnp.float32) for _ in range(NBUF)],
            pltpu.VMEM_SHARED((VOCAB * PEP, EMB), jnp.float32),
            pltpu.SemaphoreType.DMA((NBUF,)),
            pltpu.SemaphoreType.DMA((NBUF,)),
        ],
    )
    def k(comb_hbm, idx_hbm, out_hbm, idx_all, r0, r1, r2, r3, comb_sp,
          gsem, osem):
        rows = [r0, r1, r2, r3]
        wid = lax.axis_index("s") * NC + lax.axis_index("c")
        base = wid * BPW
        pl.when(lax.axis_index("s") == 0)(
            lambda: pltpu.sync_copy(comb_hbm, comb_sp))
        pltpu.sync_copy(idx_hbm.at[wid], idx_all)
        plsc.subcore_barrier()

        def wait_gather(s):
            for i in range(KB):
                pltpu.make_async_copy(
                    comb_sp.at[idx_all.at[0, pl.ds(0, PEP)]],
                    rows[s].at[i].at[pl.ds(0, PEP)], gsem.at[s]).wait()

        def wait_out(s):
            pltpu.make_async_copy(
                rows[s].at[pl.ds(0, KB), pl.ds(0, PEP)],
                out_hbm.at[pl.ds(base, KB)], osem.at[s]).wait()

        def phys(j):
            jj = j + wid * NBUF
            return jnp.where(jj >= NCHUNK, jj - NCHUNK, jj)

        def start_gather(j, s):
            jp = phys(j)
            for i in range(KB):
                pltpu.async_copy(
                    comb_sp.at[idx_all.at[jp, pl.ds(i * PEP, PEP)]],
                    rows[s].at[i].at[pl.ds(0, PEP)], gsem.at[s])

        def start_out(j, s):
            pltpu.async_copy(
                rows[s].at[pl.ds(0, KB), pl.ds(0, PEP)],
                out_hbm.at[pl.ds(base + phys(j) * KB, KB)],
                osem.at[s])

        def body(g, _):
            for s in range(NBUF):
                j = g * NBUF + s
                pl.when(g > 0)(lambda s=s: wait_out(s))
                start_gather(j, s)
                ps = (s - 1) % NBUF
                if s == 0:
                    def prev(g=g, ps=ps):
                        wait_gather(ps)
                        start_out(g * NBUF - 1, ps)
                    pl.when(g > 0)(prev)
                else:
                    wait_gather(ps)
                    start_out(j - 1, ps)
            return 0

        lax.fori_loop(0, NCHUNK // NBUF, body, 0)
        wait_gather(NBUF - 1)
        start_out(NCHUNK - 1, NBUF - 1)
        for s in range(NBUF):
            wait_out(s)

    return k(comb, idx3)


def kernel(x, aa_table, pos_table):
    x32 = x.astype(jnp.int32)
    comb, idx = _prep(x32, aa_table, pos_table)
    return _sc_gather(comb, idx.reshape(NW, NCHUNK, KROW))
